# baseline (speedup 1.0000x reference)
import functools

import jax
import jax.numpy as jnp
from jax import lax
from jax.experimental import pallas as pl
from jax.experimental.pallas import tpu as pltpu
from jax.experimental.pallas import tpu_sc as plsc

N = 10000
E = 320000
F = 128
H = 20
C = 10

NC, NS = 2, 16
NW = NC * NS
EPW = E // NW
BATCH = 125
NB = EPW // BATCH
NPAD = 10240
RPT = NPAD // NS

_mesh = plsc.VectorSubcoreMesh(
    core_axis_name="c", subcore_axis_name="s", num_cores=NC, num_subcores=NS
)


@functools.partial(
    pl.kernel,
    out_type=jax.ShapeDtypeStruct((NC, NPAD, 1), jnp.float32),
    mesh=_mesh,
    scratch_types=[
        pltpu.VMEM((NB, BATCH), jnp.int32),
        pltpu.VMEM((128, 1), jnp.float32),
        pltpu.VMEM_SHARED((NPAD, 1), jnp.float32),
        pltpu.SemaphoreType.DMA,
    ],
)
def _sc_degree(dst_hbm, zeros1_hbm, ones_hbm, out_hbm, dst_v, ones_v, deg_sh,
               sem):
    c = lax.axis_index("c")
    s = lax.axis_index("s")
    wid = s * NC + c
    pltpu.sync_copy(zeros1_hbm.at[pl.ds(s * RPT, RPT)],
                    deg_sh.at[pl.ds(s * RPT, RPT)])
    pltpu.sync_copy(ones_hbm, ones_v)
    pltpu.sync_copy(dst_hbm.at[wid], dst_v)
    plsc.subcore_barrier()

    def body(j, carry):
        pltpu.sync_copy(ones_v.at[pl.ds(0, BATCH)],
                        deg_sh.at[dst_v.at[j]], add=True)
        return carry

    lax.fori_loop(0, NB, body, 0)
    plsc.subcore_barrier()
    pltpu.sync_copy(deg_sh.at[pl.ds(s * RPT, RPT)],
                    out_hbm.at[c, pl.ds(s * RPT, RPT)])


@functools.partial(
    pl.kernel,
    out_type=jax.ShapeDtypeStruct((NC, NPAD, H), jnp.float32),
    mesh=_mesh,
    compiler_params=pltpu.CompilerParams(use_tc_tiling_on_sc=False),
    scratch_types=[
        pltpu.VMEM((NB, BATCH), jnp.int32),
        pltpu.VMEM((NB, BATCH), jnp.int32),
        pltpu.VMEM((6, BATCH, H), jnp.float32),
        pltpu.VMEM_SHARED((NPAD, H), jnp.float32),
    ] + [pltpu.SemaphoreType.DMA] * 12,
)
def _sc_aggregate(hp_hbm, src_hbm, dst_hbm, zeros_hbm, out_hbm,
                  src_v, dst_v, rows_v, acc_sh, *sems12):
    c = lax.axis_index("c")
    s = lax.axis_index("s")
    wid = s * NC + c
    pltpu.sync_copy(zeros_hbm.at[pl.ds(s * RPT, RPT)],
                    acc_sh.at[pl.ds(s * RPT, RPT)])
    pltpu.sync_copy(src_hbm.at[wid], src_v)
    pltpu.sync_copy(dst_hbm.at[wid], dst_v)
    plsc.subcore_barrier()

    D = 6
    K = 3
    gsems = sems12[:D]
    ssems = sems12[D:]
    gd = [None] * D
    sd = [None] * D
    for t in range(min(K, NB)):
        gd[t % D] = pltpu.async_copy(hp_hbm.at[src_v.at[t]],
                                     rows_v.at[t % D], gsems[t % D])
    for j in range(NB):
        b = j % D
        gd[b].wait()
        sd[b] = pltpu.async_copy(rows_v.at[b], acc_sh.at[dst_v.at[j]],
                                 ssems[b], add=True)
        jf = j + K
        if jf < NB:
            bf = jf % D
            if sd[bf] is not None:
                sd[bf].wait()
                sd[bf] = None
            gd[bf] = pltpu.async_copy(hp_hbm.at[src_v.at[jf]],
                                      rows_v.at[bf], gsems[bf])
    for b in range(D):
        if sd[b] is not None:
            sd[b].wait()
    plsc.subcore_barrier()
    pltpu.sync_copy(acc_sh.at[pl.ds(s * RPT, RPT)],
                    out_hbm.at[c, pl.ds(s * RPT, RPT)])


def _tc_mm1_body(x_ref, w1_ref, h_ref):
    h_ref[...] = jnp.dot(x_ref[...], w1_ref[...],
                         preferred_element_type=jnp.float32)


_tc_mm1 = pl.pallas_call(
    _tc_mm1_body,
    out_shape=jax.ShapeDtypeStruct((N, H), jnp.float32),
)


def _tc_scale_body(h_ref, degpair_ref, hp_ref, dis_ref):
    deg = degpair_ref[0, :N, :] + degpair_ref[1, :N, :] + 1.0
    dis = lax.rsqrt(deg)
    hp_ref[...] = h_ref[...] * dis
    dis_ref[...] = dis


_tc_scale = pl.pallas_call(
    _tc_scale_body,
    out_shape=[
        jax.ShapeDtypeStruct((N, H), jnp.float32),
        jax.ShapeDtypeStruct((N, 1), jnp.float32),
    ],
)


def _norm_relu(conv):
    nrm2 = jnp.sum(conv * conv, axis=1, keepdims=True)
    return jnp.maximum(conv * lax.rsqrt(jnp.maximum(nrm2, 1e-24)), 0.0)


def _tc_layer_body(aggpair_ref, hp_ref, dis_ref, b_ref, w_ref,
                   out_ref, hpn_ref):
    total = aggpair_ref[0, :N, :] + aggpair_ref[1, :N, :] + hp_ref[...]
    dis = dis_ref[...]
    conv = dis * total + b_ref[...]
    o = _norm_relu(conv)
    out_ref[...] = o
    hpn_ref[...] = jnp.dot(o, w_ref[...],
                           preferred_element_type=jnp.float32) * dis


_tc_layer = pl.pallas_call(
    _tc_layer_body,
    out_shape=[
        jax.ShapeDtypeStruct((N, H), jnp.float32),
        jax.ShapeDtypeStruct((N, H), jnp.float32),
    ],
)


def _tc_head_body(aggpair_ref, hp_ref, dis_ref, b_ref, out1_ref, out2_ref,
                  wlin_ref, blin_ref, fin_ref):
    total = aggpair_ref[0, :N, :] + aggpair_ref[1, :N, :] + hp_ref[...]
    conv = dis_ref[...] * total + b_ref[...]
    o3 = _norm_relu(conv)
    cat = jnp.concatenate([out1_ref[...], out2_ref[...], o3], axis=1)
    fin_ref[...] = jnp.dot(cat, wlin_ref[...],
                           preferred_element_type=jnp.float32) + blin_ref[...]


_tc_head = pl.pallas_call(
    _tc_head_body,
    out_shape=jax.ShapeDtypeStruct((N, C), jnp.float32),
)


def kernel(x, edge_index, W1, b1, W2, b2, W3, b3, Wlin, blin):
    src = edge_index[0].reshape(NW, NB, BATCH)
    dst = edge_index[1].reshape(NW, NB, BATCH)
    zeros_h = jnp.zeros((NPAD, H), jnp.float32)
    zeros_1 = jnp.zeros((NPAD, 1), jnp.float32)
    ones_h = jnp.ones((128, 1), jnp.float32)

    degpair = _sc_degree(dst, zeros_1, ones_h)
    h1 = _tc_mm1(x, W1)
    hp1, dis = _tc_scale(h1, degpair)
    agg1 = _sc_aggregate(hp1, src, dst, zeros_h)
    out1, hp2 = _tc_layer(agg1, hp1, dis, b1.reshape(1, H), W2)
    agg2 = _sc_aggregate(hp2, src, dst, zeros_h)
    out2, hp3 = _tc_layer(agg2, hp2, dis, b2.reshape(1, H), W3)
    agg3 = _sc_aggregate(hp3, src, dst, zeros_h)
    final = _tc_head(agg3, hp3, dis, b3.reshape(1, H), out1, out2,
                     Wlin, blin.reshape(1, C))
    return final


# speedup vs baseline: 1.0768x; 1.0768x over previous
import functools

import jax
import jax.numpy as jnp
from jax import lax
from jax.experimental import pallas as pl
from jax.experimental.pallas import tpu as pltpu
from jax.experimental.pallas import tpu_sc as plsc

N = 10000
E = 320000
F = 128
H = 20
C = 10

NC, NS = 2, 16
NW = NC * NS
EPW = E // NW
BATCH = 125
NB = EPW // BATCH
NPAD = 10240
RPT = NPAD // NS

_mesh = plsc.VectorSubcoreMesh(
    core_axis_name="c", subcore_axis_name="s", num_cores=NC, num_subcores=NS
)


@functools.partial(
    pl.kernel,
    out_type=jax.ShapeDtypeStruct((NC, NPAD, 1), jnp.float32),
    mesh=_mesh,
    scratch_types=[
        pltpu.VMEM((NB, BATCH), jnp.int32),
        pltpu.VMEM((128, 1), jnp.float32),
        pltpu.VMEM_SHARED((NPAD, 1), jnp.float32),
    ] + [pltpu.SemaphoreType.DMA] * 8,
)
def _sc_degree(dst_hbm, zeros1_hbm, ones_hbm, out_hbm, dst_v, ones_v, deg_sh,
               *sems8):
    c = lax.axis_index("c")
    s = lax.axis_index("s")
    wid = s * NC + c
    pltpu.sync_copy(zeros1_hbm.at[pl.ds(s * RPT, RPT)],
                    deg_sh.at[pl.ds(s * RPT, RPT)])
    pltpu.sync_copy(ones_hbm, ones_v)
    pltpu.sync_copy(dst_hbm.at[wid], dst_v)
    plsc.subcore_barrier()

    def body(j, carry):
        pltpu.sync_copy(ones_v.at[pl.ds(0, BATCH)],
                        deg_sh.at[dst_v.at[j]], add=True)
        return carry

    lax.fori_loop(0, NB, body, 0)
    plsc.subcore_barrier()
    pltpu.sync_copy(deg_sh.at[pl.ds(s * RPT, RPT)],
                    out_hbm.at[c, pl.ds(s * RPT, RPT)])


@functools.partial(
    pl.kernel,
    out_type=jax.ShapeDtypeStruct((NC, NPAD, H), jnp.float32),
    mesh=_mesh,
    compiler_params=pltpu.CompilerParams(use_tc_tiling_on_sc=False),
    scratch_types=[
        pltpu.VMEM((NB, BATCH), jnp.int32),
        pltpu.VMEM((NB, BATCH), jnp.int32),
        pltpu.VMEM((6, BATCH, H), jnp.float32),
        pltpu.VMEM_SHARED((NPAD, H), jnp.float32),
        pltpu.VMEM_SHARED((NPAD, H), jnp.float32),
    ] + [pltpu.SemaphoreType.DMA] * 12,
)
def _sc_aggregate(hp_hbm, src_hbm, dst_hbm, zeros_hbm, out_hbm,
                  src_v, dst_v, rows_v, acc_sh, hp_sh, *sems12):
    c = lax.axis_index("c")
    s = lax.axis_index("s")
    wid = s * NC + c
    pltpu.sync_copy(zeros_hbm.at[pl.ds(s * RPT, RPT)],
                    acc_sh.at[pl.ds(s * RPT, RPT)])
    pltpu.sync_copy(hp_hbm.at[pl.ds(s * RPT, RPT)],
                    hp_sh.at[pl.ds(s * RPT, RPT)])
    pltpu.sync_copy(src_hbm.at[wid], src_v)
    pltpu.sync_copy(dst_hbm.at[wid], dst_v)
    plsc.subcore_barrier()

    D = 6
    K = 3
    gsems = sems12[:D]
    ssems = sems12[D:]
    gd = [None] * D
    sd = [None] * D
    for t in range(min(K, NB)):
        gd[t % D] = pltpu.async_copy(hp_sh.at[src_v.at[t]],
                                     rows_v.at[t % D], gsems[t % D])
    for j in range(NB):
        b = j % D
        gd[b].wait()
        sd[b] = pltpu.async_copy(rows_v.at[b], acc_sh.at[dst_v.at[j]],
                                 ssems[b], add=True)
        jf = j + K
        if jf < NB:
            bf = jf % D
            if sd[bf] is not None:
                sd[bf].wait()
                sd[bf] = None
            gd[bf] = pltpu.async_copy(hp_sh.at[src_v.at[jf]],
                                      rows_v.at[bf], gsems[bf])
    for b in range(D):
        if sd[b] is not None:
            sd[b].wait()
    plsc.subcore_barrier()
    pltpu.sync_copy(acc_sh.at[pl.ds(s * RPT, RPT)],
                    out_hbm.at[c, pl.ds(s * RPT, RPT)])


def _tc_mm1_body(x_ref, w1_ref, h_ref):
    h_ref[...] = jnp.dot(x_ref[...], w1_ref[...],
                         preferred_element_type=jnp.float32)


_tc_mm1 = pl.pallas_call(
    _tc_mm1_body,
    out_shape=jax.ShapeDtypeStruct((N, H), jnp.float32),
)


def _tc_scale_body(h_ref, degpair_ref, hp_ref, dis_ref):
    deg = degpair_ref[0, :N, :] + degpair_ref[1, :N, :] + 1.0
    dis = lax.rsqrt(deg)
    hp_ref[...] = jnp.concatenate(
        [h_ref[...] * dis, jnp.zeros((NPAD - N, H), jnp.float32)], axis=0)
    dis_ref[...] = dis


_tc_scale = pl.pallas_call(
    _tc_scale_body,
    out_shape=[
        jax.ShapeDtypeStruct((NPAD, H), jnp.float32),
        jax.ShapeDtypeStruct((N, 1), jnp.float32),
    ],
)


def _norm_relu(conv):
    nrm2 = jnp.sum(conv * conv, axis=1, keepdims=True)
    return jnp.maximum(conv * lax.rsqrt(jnp.maximum(nrm2, 1e-24)), 0.0)


def _tc_layer_body(aggpair_ref, hp_ref, dis_ref, b_ref, w_ref,
                   out_ref, hpn_ref):
    total = (aggpair_ref[0, :N, :] + aggpair_ref[1, :N, :]
             + hp_ref[:N, :])
    dis = dis_ref[...]
    conv = dis * total + b_ref[...]
    o = _norm_relu(conv)
    out_ref[...] = o
    hpn = jnp.dot(o, w_ref[...], preferred_element_type=jnp.float32) * dis
    hpn_ref[...] = jnp.concatenate(
        [hpn, jnp.zeros((NPAD - N, H), jnp.float32)], axis=0)


_tc_layer = pl.pallas_call(
    _tc_layer_body,
    out_shape=[
        jax.ShapeDtypeStruct((N, H), jnp.float32),
        jax.ShapeDtypeStruct((NPAD, H), jnp.float32),
    ],
)


def _tc_head_body(aggpair_ref, hp_ref, dis_ref, b_ref, out1_ref, out2_ref,
                  wlin_ref, blin_ref, fin_ref):
    total = (aggpair_ref[0, :N, :] + aggpair_ref[1, :N, :]
             + hp_ref[:N, :])
    conv = dis_ref[...] * total + b_ref[...]
    o3 = _norm_relu(conv)
    cat = jnp.concatenate([out1_ref[...], out2_ref[...], o3], axis=1)
    fin_ref[...] = jnp.dot(cat, wlin_ref[...],
                           preferred_element_type=jnp.float32) + blin_ref[...]


_tc_head = pl.pallas_call(
    _tc_head_body,
    out_shape=jax.ShapeDtypeStruct((N, C), jnp.float32),
)


def kernel(x, edge_index, W1, b1, W2, b2, W3, b3, Wlin, blin):
    src = edge_index[0].reshape(NW, NB, BATCH)
    dst = edge_index[1].reshape(NW, NB, BATCH)
    zeros_h = jnp.zeros((NPAD, H), jnp.float32)
    zeros_1 = jnp.zeros((NPAD, 1), jnp.float32)
    ones_h = jnp.ones((128, 1), jnp.float32)

    degpair = _sc_degree(dst, zeros_1, ones_h)
    h1 = _tc_mm1(x, W1)
    hp1, dis = _tc_scale(h1, degpair)
    agg1 = _sc_aggregate(hp1, src, dst, zeros_h)
    out1, hp2 = _tc_layer(agg1, hp1, dis, b1.reshape(1, H), W2)
    agg2 = _sc_aggregate(hp2, src, dst, zeros_h)
    out2, hp3 = _tc_layer(agg2, hp2, dis, b2.reshape(1, H), W3)
    agg3 = _sc_aggregate(hp3, src, dst, zeros_h)
    final = _tc_head(agg3, hp3, dis, b3.reshape(1, H), out1, out2,
                     Wlin, blin.reshape(1, C))
    return final
